# baseline (speedup 1.0000x reference)
import functools

import jax
import jax.numpy as jnp
from jax import lax
from jax.experimental import pallas as pl
from jax.experimental.pallas import tpu as pltpu
from jax.experimental.pallas import tpu_sc as plsc

_SC_PARAMS = pltpu.CompilerParams(use_tc_tiling_on_sc=False,
                                  internal_scratch_in_bytes=256 * 1024)

NC = 2
NS = 16
NW = NC * NS
LANE = 128
IDX_BLK = 8
TILE = 2000


def _round_up(x, m):
    return (x + m - 1) // m * m


def _acc_geometry(n):
    slab = _round_up(-(-(n + 1) // NS), 64)
    zch = next(z for z in range(128, 7, -8) if slab % z == 0)
    return NS * slab, slab, zch


def _sc_gather_pass(g_tbl, idx_g, idx_s, n, width, ib):
    acc_rows, slab, zch = _acc_geometry(n)
    n_zch = slab // zch
    w_per = idx_g.shape[0] // NW
    n_blk = w_per // ib
    zeros = jnp.zeros((zch, width), jnp.float32)
    mesh = plsc.VectorSubcoreMesh(core_axis_name="c", subcore_axis_name="s")

    @functools.partial(
        pl.kernel, mesh=mesh, compiler_params=_SC_PARAMS,
        out_type=jax.ShapeDtypeStruct((NC, acc_rows, width), jnp.float32),
        scratch_types=[
            pltpu.VMEM_SHARED((acc_rows, width), jnp.float32),
            pltpu.VMEM((ib, LANE), jnp.int32),
            pltpu.VMEM((ib, LANE), jnp.int32),
            pltpu.VMEM((ib, LANE, width), jnp.float32),
            pltpu.VMEM((zch, width), jnp.float32),
            pltpu.SemaphoreType.DMA,
            pltpu.SemaphoreType.DMA,
        ])
    def run(g_hbm, ig_hbm, is_hbm, z_hbm, out_hbm,
            acc, igv, isv, rowsb, zbuf, semg, sems):
        cid = lax.axis_index("c")
        sid = lax.axis_index("s")
        wid = sid * NC + cid
        pltpu.sync_copy(z_hbm, zbuf)

        @pl.loop(0, n_zch)
        def _(k):
            pltpu.sync_copy(zbuf, acc.at[pl.ds((sid * n_zch + k) * zch, zch)])

        plsc.subcore_barrier()
        wbase = wid * w_per

        @pl.loop(0, n_blk)
        def _(t):
            pltpu.sync_copy(ig_hbm.at[pl.ds(wbase + t * ib, ib)], igv)
            pltpu.sync_copy(is_hbm.at[pl.ds(wbase + t * ib, ib)], isv)
            gd = [pltpu.async_copy(g_hbm.at[igv.at[j]], rowsb.at[j], semg)
                  for j in range(ib)]
            for d in gd:
                d.wait()
            sd = [pltpu.async_copy(rowsb.at[j], acc.at[isv.at[j]], sems, add=True)
                  for j in range(ib)]
            for d in sd:
                d.wait()

        plsc.subcore_barrier()

        @pl.loop(0, n_zch)
        def _(k):
            base = (sid * n_zch + k) * zch
            pltpu.sync_copy(acc.at[pl.ds(base, zch)],
                            out_hbm.at[cid, pl.ds(base, zch)])

    return run(g_tbl, idx_g, idx_s, zeros)


def _sc_count_pass(const_rows, idx_s, n, width, ib):
    acc_rows, slab, zch = _acc_geometry(n)
    n_zch = slab // zch
    w_per = idx_s.shape[0] // NW
    n_blk = w_per // ib
    zeros = jnp.zeros((zch, width), jnp.float32)
    mesh = plsc.VectorSubcoreMesh(core_axis_name="c", subcore_axis_name="s")

    @functools.partial(
        pl.kernel, mesh=mesh, compiler_params=_SC_PARAMS,
        out_type=jax.ShapeDtypeStruct((NC, acc_rows, width), jnp.float32),
        scratch_types=[
            pltpu.VMEM_SHARED((acc_rows, width), jnp.float32),
            pltpu.VMEM((ib, LANE), jnp.int32),
            pltpu.VMEM((LANE, width), jnp.float32),
            pltpu.VMEM((zch, width), jnp.float32),
            pltpu.SemaphoreType.DMA,
        ])
    def run(c_hbm, is_hbm, z_hbm, out_hbm, acc, isv, rowsb, zbuf, sems):
        cid = lax.axis_index("c")
        sid = lax.axis_index("s")
        wid = sid * NC + cid
        pltpu.sync_copy(z_hbm, zbuf)
        pltpu.sync_copy(c_hbm, rowsb)

        @pl.loop(0, n_zch)
        def _(k):
            pltpu.sync_copy(zbuf, acc.at[pl.ds((sid * n_zch + k) * zch, zch)])

        plsc.subcore_barrier()
        wbase = wid * w_per

        @pl.loop(0, n_blk)
        def _(t):
            pltpu.sync_copy(is_hbm.at[pl.ds(wbase + t * ib, ib)], isv)
            sd = [pltpu.async_copy(rowsb, acc.at[isv.at[j]], sems, add=True)
                  for j in range(ib)]
            for d in sd:
                d.wait()

        plsc.subcore_barrier()

        @pl.loop(0, n_zch)
        def _(k):
            base = (sid * n_zch + k) * zch
            pltpu.sync_copy(acc.at[pl.ds(base, zch)],
                            out_hbm.at[cid, pl.ds(base, zch)])

    return run(const_rows, idx_s, zeros)


def _pad_idx(idx, pad_val):
    e = idx.shape[0]
    rows = _round_up(-(-e // LANE), NC * NS * 16)
    pad = rows * LANE - e
    return jnp.concatenate(
        [idx.astype(jnp.int32),
         jnp.full((pad,), pad_val, jnp.int32)]).reshape(rows, LANE)


def _row_spec(f):
    return pl.BlockSpec((TILE, f), lambda i: (i, 0))


def _row3_spec(a, b):
    return pl.BlockSpec((TILE, a, b), lambda i: (i, 0, 0))


def _part_spec(acc_rows, f):
    return pl.BlockSpec((NC, TILE, f), lambda i: (0, i, 0))


def _full_spec(shape):
    nd = len(shape)
    return pl.BlockSpec(shape, lambda i, _n=nd: (0,) * _n)


def _tc_call(body, n, in_arrays, in_specs, out_shapes, out_specs):
    grid = n // TILE
    return pl.pallas_call(
        body, grid=(grid,), in_specs=in_specs, out_specs=out_specs,
        out_shape=out_shapes)(*in_arrays)


def _pad16(x):
    return jnp.concatenate(
        [x, jnp.zeros((x.shape[0], 16 - x.shape[1]), x.dtype)], axis=1)


def _stage_a(p_feat, W1, d_feat, Wd, bd, n):
    def body(pf, w1, df, wd, bdr, xw1_o, d0_o):
        xw1_o[...] = _pad16(jnp.dot(pf[...], w1[...],
                                    preferred_element_type=jnp.float32))
        d0 = jnp.dot(df[...], wd[...], preferred_element_type=jnp.float32)
        d0_o[...] = _pad16(jnp.maximum(d0 + bdr[...], 0.0))
    return _tc_call(
        body, n,
        [p_feat, W1, d_feat, Wd, bd.reshape(1, -1)],
        [_row_spec(128), _full_spec(W1.shape), _row_spec(128),
         _full_spec(Wd.shape), _full_spec((1, bd.shape[0]))],
        [jax.ShapeDtypeStruct((n, 16), jnp.float32),
         jax.ShapeDtypeStruct((n, 16), jnp.float32)],
        [_row_spec(16), _row_spec(16)])


def _dinv_of(degp):
    deg = degp[0, :, 0:1] + degp[1, :, 0:1] + 1.0
    return lax.rsqrt(deg)


def _stage_b(degp, xw1, n, acc_rows):
    def body(dp, xw, y1_o):
        y1_o[...] = _dinv_of(dp[...]) * xw[...]
    return _tc_call(
        body, n, [degp, xw1], [_part_spec(acc_rows, 16), _row_spec(16)],
        jax.ShapeDtypeStruct((n, 16), jnp.float32), _row_spec(16))


def _stage_c(degp, agg1p, xw1, W2, b1, n, acc_rows):
    def body(dp, ap, xw, w2, b1r, y2_o, xw2_o):
        dinv = _dinv_of(dp[...])
        agg = ap[0] + ap[1]
        h1 = jnp.maximum(dinv * agg + dinv * dinv * xw[...] + b1r[...], 0.0)
        xw2 = _pad16(jnp.dot(h1[:, :10], w2[...],
                             preferred_element_type=jnp.float32))
        xw2_o[...] = xw2
        y2_o[...] = dinv * xw2
    b1p = _pad16(b1.reshape(1, -1))
    return _tc_call(
        body, n, [degp, agg1p, xw1, W2, b1p],
        [_part_spec(acc_rows, 16), _part_spec(acc_rows, 16), _row_spec(16),
         _full_spec(W2.shape), _full_spec((1, 16))],
        [jax.ShapeDtypeStruct((n, 16), jnp.float32),
         jax.ShapeDtypeStruct((n, 16), jnp.float32)],
        [_row_spec(16), _row_spec(16)])


def _stage_d(degp, agg2p, xw2, b2, rpd_comp, rpd_basis2d, n, acc_rows):
    def body(dp, ap, xw, b2r, comp, basis, p2_o, y3_o):
        dinv = _dinv_of(dp[...])
        agg = ap[0] + ap[1]
        p2 = jnp.maximum(dinv * agg + dinv * dinv * xw[...] + b2r[...], 0.0)
        p2_o[...] = p2
        wpd = comp[0, 0] * basis[...]
        zw = jnp.dot(p2[:, :5], wpd, preferred_element_type=jnp.float32)
        y3_o[...] = jnp.concatenate(
            [zw, jnp.ones((TILE, 1), jnp.float32),
             jnp.zeros((TILE, 11), jnp.float32)], axis=1)
    b2p = _pad16(b2.reshape(1, -1))
    return _tc_call(
        body, n, [degp, agg2p, xw2, b2p, rpd_comp, rpd_basis2d],
        [_part_spec(acc_rows, 16), _part_spec(acc_rows, 16), _row_spec(16),
         _full_spec((1, 16)), _full_spec((1, 1)), _full_spec((5, 4))],
        [jax.ShapeDtypeStruct((n, 16), jnp.float32),
         jax.ShapeDtypeStruct((n, 16), jnp.float32)],
        [_row_spec(16), _row_spec(16)])


def _rel_rows(a, r, count_col=7):
    f = a.shape[1]
    parts = [jnp.zeros((a.shape[0], 8 * r), jnp.float32), a,
             jnp.zeros((a.shape[0], count_col - f), jnp.float32),
             jnp.ones((a.shape[0], 1), jnp.float32),
             jnp.zeros((a.shape[0], 24 - 8 * r), jnp.float32)]
    return jnp.concatenate([p for p in parts if p.shape[1] > 0], axis=1)


def _stage_e(agg3p, p2, rpd_root, rpd_bias, d0, rdd1_comp, rdd1_basis,
             n, acc_rows):
    def body(ap, p2r, root, biasr, d0r, comp, basis, x10_o, g1_o):
        agg = ap[0] + ap[1]
        cnt = jnp.maximum(agg[:, 4:5], 1.0)
        p3 = jnp.maximum(
            jnp.dot(p2r[:, :5], root[...], preferred_element_type=jnp.float32)
            + biasr[...] + agg[:, :4] / cnt, 0.0)
        x10 = jnp.concatenate([d0r[:, :6], p3], axis=1)
        x10_o[...] = _pad16(x10)
        w = jnp.einsum('rb,bio->rio', comp[...], basis[...],
                       preferred_element_type=jnp.float32)
        rows = [_rel_rows(jnp.dot(x10, w[r],
                                  preferred_element_type=jnp.float32), r)
                for r in range(3)]
        g1_o[...] = jnp.stack(rows, axis=1)
    return _tc_call(
        body, n,
        [agg3p, p2, rpd_root, rpd_bias.reshape(1, -1), d0, rdd1_comp,
         rdd1_basis],
        [_part_spec(acc_rows, 16), _row_spec(16), _full_spec((5, 4)),
         _full_spec((1, 4)), _row_spec(16), _full_spec((3, 3)),
         _full_spec((3, 10, 7))],
        [jax.ShapeDtypeStruct((n, 16), jnp.float32),
         jax.ShapeDtypeStruct((n, 3, 32), jnp.float32)],
        [_row_spec(16), _row3_spec(3, 32)])


def _rgcn_mix(acc, x, root, bias, f_out):
    out = jnp.dot(x, root, preferred_element_type=jnp.float32) + bias
    for r in range(3):
        cnt = jnp.maximum(acc[:, 8 * r + 7:8 * r + 8], 1.0)
        out = out + acc[:, 8 * r:8 * r + f_out] / cnt
    return out


def _stage_f(acc1p, x10, rdd1_root, rdd1_bias, rdd2_comp, rdd2_basis,
             n, acc_rows):
    def body(ap, x10r, root, biasr, comp, basis, d1_o, g2_o):
        acc = ap[0] + ap[1]
        d1 = jnp.maximum(_rgcn_mix(acc, x10r[:, :10], root[...], biasr[...], 7),
                         0.0)
        d1_o[...] = _pad16(d1)
        w = jnp.einsum('rb,bio->rio', comp[...], basis[...],
                       preferred_element_type=jnp.float32)
        rows = [_rel_rows(jnp.dot(d1, w[r],
                                  preferred_element_type=jnp.float32), r)
                for r in range(3)]
        g2_o[...] = jnp.stack(rows, axis=1)
    return _tc_call(
        body, n,
        [acc1p, x10, rdd1_root, rdd1_bias.reshape(1, -1), rdd2_comp,
         rdd2_basis],
        [_part_spec(acc_rows, 32), _row_spec(16), _full_spec((10, 7)),
         _full_spec((1, 7)), _full_spec((3, 3)), _full_spec((3, 7, 5))],
        [jax.ShapeDtypeStruct((n, 16), jnp.float32),
         jax.ShapeDtypeStruct((n, 3, 32), jnp.float32)],
        [_row_spec(16), _row3_spec(3, 32)])


def _stage_g(acc2p, d1, rdd2_root, rdd2_bias, n, acc_rows):
    def body(ap, d1r, root, biasr, out_o):
        acc = ap[0] + ap[1]
        out_o[...] = _rgcn_mix(acc, d1r[:, :7], root[...], biasr[...], 5)
    return _tc_call(
        body, n, [acc2p, d1, rdd2_root, rdd2_bias.reshape(1, -1)],
        [_part_spec(acc_rows, 32), _row_spec(16), _full_spec((7, 5)),
         _full_spec((1, 5))],
        jax.ShapeDtypeStruct((n, 5), jnp.float32), _row_spec(5))


def kernel(p_feat, d_feat, pp_edge_ind, pd_edge_ind, dd_edge_ind, dd_edge_type,
           W1, b1, W2, b2, rpd_basis, rpd_comp, rpd_root, rpd_bias,
           Wd, bd, rdd1_basis, rdd1_comp, rdd1_root, rdd1_bias,
           rdd2_basis, rdd2_comp, rdd2_root, rdd2_bias):
    n = p_feat.shape[0]
    acc_rows, _, _ = _acc_geometry(n)
    junk = n

    pp_src = _pad_idx(pp_edge_ind[0], 0)
    pp_dst = _pad_idx(pp_edge_ind[1], junk)
    pd_src = _pad_idx(pd_edge_ind[0], 0)
    pd_dst = _pad_idx(pd_edge_ind[1], junk)
    dd_g = _pad_idx(3 * dd_edge_ind[0] + dd_edge_type, 0)
    dd_dst = _pad_idx(dd_edge_ind[1], junk)

    one_rows = jnp.concatenate(
        [jnp.ones((LANE, 1), jnp.float32),
         jnp.zeros((LANE, 15), jnp.float32)], axis=1)

    degp = _sc_count_pass(one_rows, pp_dst, n, 16, 16)
    xw1, d0 = _stage_a(p_feat, W1, d_feat, Wd, bd, n)

    y1 = _stage_b(degp, xw1, n, acc_rows)
    agg1p = _sc_gather_pass(y1, pp_src, pp_dst, n, 16, 16)

    y2, xw2 = _stage_c(degp, agg1p, xw1, W2, b1, n, acc_rows)
    agg2p = _sc_gather_pass(y2, pp_src, pp_dst, n, 16, 16)

    p2, y3 = _stage_d(degp, agg2p, xw2, b2, rpd_comp,
                      rpd_basis.reshape(5, 4), n, acc_rows)
    agg3p = _sc_gather_pass(y3, pd_src, pd_dst, n, 16, 16)

    x10, g1 = _stage_e(agg3p, p2, rpd_root, rpd_bias, d0,
                       rdd1_comp, rdd1_basis, n, acc_rows)
    acc1p = _sc_gather_pass(g1.reshape(3 * n, 32), dd_g, dd_dst, n, 32, 4)

    d1, g2 = _stage_f(acc1p, x10, rdd1_root, rdd1_bias,
                      rdd2_comp, rdd2_basis, n, acc_rows)
    acc2p = _sc_gather_pass(g2.reshape(3 * n, 32), dd_g, dd_dst, n, 32, 4)

    return _stage_g(acc2p, d1, rdd2_root, rdd2_bias, n, acc_rows)


# speedup vs baseline: 1.5700x; 1.5700x over previous
import functools

import jax
import jax.numpy as jnp
from jax import lax
from jax.experimental import pallas as pl
from jax.experimental.pallas import tpu as pltpu
from jax.experimental.pallas import tpu_sc as plsc

_SC_PARAMS = pltpu.CompilerParams(use_tc_tiling_on_sc=False,
                                  internal_scratch_in_bytes=256 * 1024)

NC = 2
NS = 16
NW = NC * NS
LANE = 128
IDX_BLK = 8
TILE = 2000


def _round_up(x, m):
    return (x + m - 1) // m * m


def _acc_geometry(n):
    slab = _round_up(-(-(n + 1) // NS), 64)
    zch = next(z for z in range(128, 7, -8) if slab % z == 0)
    return NS * slab, slab, zch


def _sc_gather_pass(g_tbl, idx_g, idx_s, n, width, ib):
    acc_rows, slab, zch = _acc_geometry(n)
    n_zch = slab // zch
    w_per = idx_g.shape[0] // NW
    n_blk = w_per // ib
    zeros = jnp.zeros((zch, width), jnp.float32)
    mesh = plsc.VectorSubcoreMesh(core_axis_name="c", subcore_axis_name="s")

    @functools.partial(
        pl.kernel, mesh=mesh, compiler_params=_SC_PARAMS,
        out_type=jax.ShapeDtypeStruct((NC, acc_rows, width), jnp.float32),
        scratch_types=[
            pltpu.VMEM_SHARED((acc_rows, width), jnp.float32),
            pltpu.VMEM((ib, LANE), jnp.int32),
            pltpu.VMEM((ib, LANE), jnp.int32),
            pltpu.VMEM((ib, LANE, width), jnp.float32),
            pltpu.VMEM((zch, width), jnp.float32),
            pltpu.SemaphoreType.DMA,
            pltpu.SemaphoreType.DMA,
        ])
    def run(g_hbm, ig_hbm, is_hbm, z_hbm, out_hbm,
            acc, igv, isv, rowsb, zbuf, semg, sems):
        cid = lax.axis_index("c")
        sid = lax.axis_index("s")
        wid = sid * NC + cid
        pltpu.sync_copy(z_hbm, zbuf)

        @pl.loop(0, n_zch)
        def _(k):
            pltpu.sync_copy(zbuf, acc.at[pl.ds((sid * n_zch + k) * zch, zch)])

        plsc.subcore_barrier()
        wbase = wid * w_per

        @pl.loop(0, n_blk)
        def _(t):
            pltpu.sync_copy(ig_hbm.at[pl.ds(wbase + t * ib, ib)], igv)
            pltpu.sync_copy(is_hbm.at[pl.ds(wbase + t * ib, ib)], isv)
            gd = [pltpu.async_copy(g_hbm.at[igv.at[j]], rowsb.at[j], semg)
                  for j in range(ib)]
            for d in gd:
                d.wait()
            sd = [pltpu.async_copy(rowsb.at[j], acc.at[isv.at[j]], sems, add=True)
                  for j in range(ib)]
            for d in sd:
                d.wait()

        plsc.subcore_barrier()

        @pl.loop(0, n_zch)
        def _(k):
            base = (sid * n_zch + k) * zch
            pltpu.sync_copy(acc.at[pl.ds(base, zch)],
                            out_hbm.at[cid, pl.ds(base, zch)])

    return run(g_tbl, idx_g, idx_s, zeros)


def _sc_count_pass(const_rows, idx_s, n, width, ib):
    acc_rows, slab, zch = _acc_geometry(n)
    n_zch = slab // zch
    w_per = idx_s.shape[0] // NW
    n_blk = w_per // ib
    zeros = jnp.zeros((zch, width), jnp.float32)
    mesh = plsc.VectorSubcoreMesh(core_axis_name="c", subcore_axis_name="s")

    @functools.partial(
        pl.kernel, mesh=mesh, compiler_params=_SC_PARAMS,
        out_type=jax.ShapeDtypeStruct((NC, acc_rows, width), jnp.float32),
        scratch_types=[
            pltpu.VMEM_SHARED((acc_rows, width), jnp.float32),
            pltpu.VMEM((ib, LANE), jnp.int32),
            pltpu.VMEM((LANE, width), jnp.float32),
            pltpu.VMEM((zch, width), jnp.float32),
            pltpu.SemaphoreType.DMA,
        ])
    def run(c_hbm, is_hbm, z_hbm, out_hbm, acc, isv, rowsb, zbuf, sems):
        cid = lax.axis_index("c")
        sid = lax.axis_index("s")
        wid = sid * NC + cid
        pltpu.sync_copy(z_hbm, zbuf)
        pltpu.sync_copy(c_hbm, rowsb)

        @pl.loop(0, n_zch)
        def _(k):
            pltpu.sync_copy(zbuf, acc.at[pl.ds((sid * n_zch + k) * zch, zch)])

        plsc.subcore_barrier()
        wbase = wid * w_per

        @pl.loop(0, n_blk)
        def _(t):
            pltpu.sync_copy(is_hbm.at[pl.ds(wbase + t * ib, ib)], isv)
            sd = [pltpu.async_copy(rowsb, acc.at[isv.at[j]], sems, add=True)
                  for j in range(ib)]
            for d in sd:
                d.wait()

        plsc.subcore_barrier()

        @pl.loop(0, n_zch)
        def _(k):
            base = (sid * n_zch + k) * zch
            pltpu.sync_copy(acc.at[pl.ds(base, zch)],
                            out_hbm.at[cid, pl.ds(base, zch)])

    return run(const_rows, idx_s, zeros)


def _pad_idx(idx, pad_base, pad_num):
    e = idx.shape[0]
    rows = _round_up(-(-e // LANE), NC * NS * IDX_BLK)
    pad = rows * LANE - e
    fill = pad_base + jnp.arange(pad, dtype=jnp.int32) % pad_num
    return jnp.concatenate([idx.astype(jnp.int32), fill]).reshape(rows, LANE)


def _row_spec(f):
    return pl.BlockSpec((TILE, f), lambda i: (i, 0))


def _row3_spec(a, b):
    return pl.BlockSpec((TILE, a, b), lambda i: (i, 0, 0))


def _part_spec(acc_rows, f):
    return pl.BlockSpec((NC, TILE, f), lambda i: (0, i, 0))


def _full_spec(shape):
    nd = len(shape)
    return pl.BlockSpec(shape, lambda i, _n=nd: (0,) * _n)


def _tc_call(body, n, in_arrays, in_specs, out_shapes, out_specs):
    grid = n // TILE
    return pl.pallas_call(
        body, grid=(grid,), in_specs=in_specs, out_specs=out_specs,
        out_shape=out_shapes)(*in_arrays)


def _pad16(x):
    return jnp.concatenate(
        [x, jnp.zeros((x.shape[0], 16 - x.shape[1]), x.dtype)], axis=1)


def _stage_a(p_feat, W1, d_feat, Wd, bd, n):
    def body(pf, w1, df, wd, bdr, xw1_o, d0_o):
        xw1_o[...] = _pad16(jnp.dot(pf[...], w1[...],
                                    preferred_element_type=jnp.float32))
        d0 = jnp.dot(df[...], wd[...], preferred_element_type=jnp.float32)
        d0_o[...] = _pad16(jnp.maximum(d0 + bdr[...], 0.0))
    return _tc_call(
        body, n,
        [p_feat, W1, d_feat, Wd, bd.reshape(1, -1)],
        [_row_spec(128), _full_spec(W1.shape), _row_spec(128),
         _full_spec(Wd.shape), _full_spec((1, bd.shape[0]))],
        [jax.ShapeDtypeStruct((n, 16), jnp.float32),
         jax.ShapeDtypeStruct((n, 16), jnp.float32)],
        [_row_spec(16), _row_spec(16)])


def _dinv_of(degp):
    deg = degp[0, :, 0:1] + degp[1, :, 0:1] + 1.0
    return lax.rsqrt(deg)


def _stage_b(degp, xw1, n, acc_rows):
    def body(dp, xw, y1_o):
        y1_o[...] = _dinv_of(dp[...]) * xw[...]
    return _tc_call(
        body, n, [degp, xw1], [_part_spec(acc_rows, 16), _row_spec(16)],
        jax.ShapeDtypeStruct((n, 16), jnp.float32), _row_spec(16))


def _stage_c(degp, agg1p, xw1, W2, b1, n, acc_rows):
    def body(dp, ap, xw, w2, b1r, y2_o, xw2_o):
        dinv = _dinv_of(dp[...])
        agg = ap[0] + ap[1]
        h1 = jnp.maximum(dinv * agg + dinv * dinv * xw[...] + b1r[...], 0.0)
        xw2 = _pad16(jnp.dot(h1[:, :10], w2[...],
                             preferred_element_type=jnp.float32))
        xw2_o[...] = xw2
        y2_o[...] = dinv * xw2
    b1p = _pad16(b1.reshape(1, -1))
    return _tc_call(
        body, n, [degp, agg1p, xw1, W2, b1p],
        [_part_spec(acc_rows, 16), _part_spec(acc_rows, 16), _row_spec(16),
         _full_spec(W2.shape), _full_spec((1, 16))],
        [jax.ShapeDtypeStruct((n, 16), jnp.float32),
         jax.ShapeDtypeStruct((n, 16), jnp.float32)],
        [_row_spec(16), _row_spec(16)])


def _stage_d(degp, agg2p, xw2, b2, rpd_comp, rpd_basis2d, n, acc_rows):
    def body(dp, ap, xw, b2r, comp, basis, p2_o, y3_o):
        dinv = _dinv_of(dp[...])
        agg = ap[0] + ap[1]
        p2 = jnp.maximum(dinv * agg + dinv * dinv * xw[...] + b2r[...], 0.0)
        p2_o[...] = p2
        wpd = comp[0, 0] * basis[...]
        zw = jnp.dot(p2[:, :5], wpd, preferred_element_type=jnp.float32)
        y3_o[...] = jnp.concatenate(
            [zw, jnp.ones((TILE, 1), jnp.float32),
             jnp.zeros((TILE, 11), jnp.float32)], axis=1)
    b2p = _pad16(b2.reshape(1, -1))
    return _tc_call(
        body, n, [degp, agg2p, xw2, b2p, rpd_comp, rpd_basis2d],
        [_part_spec(acc_rows, 16), _part_spec(acc_rows, 16), _row_spec(16),
         _full_spec((1, 16)), _full_spec((1, 1)), _full_spec((5, 4))],
        [jax.ShapeDtypeStruct((n, 16), jnp.float32),
         jax.ShapeDtypeStruct((n, 16), jnp.float32)],
        [_row_spec(16), _row_spec(16)])


def _rel_rows(a, r, count_col=7):
    f = a.shape[1]
    parts = [jnp.zeros((a.shape[0], 8 * r), jnp.float32), a,
             jnp.zeros((a.shape[0], count_col - f), jnp.float32),
             jnp.ones((a.shape[0], 1), jnp.float32),
             jnp.zeros((a.shape[0], 24 - 8 * r), jnp.float32)]
    return jnp.concatenate([p for p in parts if p.shape[1] > 0], axis=1)


def _stage_e(agg3p, p2, rpd_root, rpd_bias, d0, rdd1_comp, rdd1_basis,
             n, acc_rows):
    def body(ap, p2r, root, biasr, d0r, comp, basis, x10_o, g1_o):
        agg = ap[0] + ap[1]
        cnt = jnp.maximum(agg[:, 4:5], 1.0)
        p3 = jnp.maximum(
            jnp.dot(p2r[:, :5], root[...], preferred_element_type=jnp.float32)
            + biasr[...] + agg[:, :4] / cnt, 0.0)
        x10 = jnp.concatenate([d0r[:, :6], p3], axis=1)
        x10_o[...] = _pad16(x10)
        w = jnp.einsum('rb,bio->rio', comp[...], basis[...],
                       preferred_element_type=jnp.float32)
        rows = [_rel_rows(jnp.dot(x10, w[r],
                                  preferred_element_type=jnp.float32), r)
                for r in range(3)]
        g1_o[...] = jnp.stack(rows, axis=1)
    return _tc_call(
        body, n,
        [agg3p, p2, rpd_root, rpd_bias.reshape(1, -1), d0, rdd1_comp,
         rdd1_basis],
        [_part_spec(acc_rows, 16), _row_spec(16), _full_spec((5, 4)),
         _full_spec((1, 4)), _row_spec(16), _full_spec((3, 3)),
         _full_spec((3, 10, 7))],
        [jax.ShapeDtypeStruct((n, 16), jnp.float32),
         jax.ShapeDtypeStruct((n, 3, 32), jnp.float32)],
        [_row_spec(16), _row3_spec(3, 32)])


def _rgcn_mix(acc, x, root, bias, f_out):
    out = jnp.dot(x, root, preferred_element_type=jnp.float32) + bias
    for r in range(3):
        cnt = jnp.maximum(acc[:, 8 * r + 7:8 * r + 8], 1.0)
        out = out + acc[:, 8 * r:8 * r + f_out] / cnt
    return out


def _stage_f(acc1p, x10, rdd1_root, rdd1_bias, rdd2_comp, rdd2_basis,
             n, acc_rows):
    def body(ap, x10r, root, biasr, comp, basis, d1_o, g2_o):
        acc = ap[0] + ap[1]
        d1 = jnp.maximum(_rgcn_mix(acc, x10r[:, :10], root[...], biasr[...], 7),
                         0.0)
        d1_o[...] = _pad16(d1)
        w = jnp.einsum('rb,bio->rio', comp[...], basis[...],
                       preferred_element_type=jnp.float32)
        rows = [_rel_rows(jnp.dot(d1, w[r],
                                  preferred_element_type=jnp.float32), r)
                for r in range(3)]
        g2_o[...] = jnp.stack(rows, axis=1)
    return _tc_call(
        body, n,
        [acc1p, x10, rdd1_root, rdd1_bias.reshape(1, -1), rdd2_comp,
         rdd2_basis],
        [_part_spec(acc_rows, 32), _row_spec(16), _full_spec((10, 7)),
         _full_spec((1, 7)), _full_spec((3, 3)), _full_spec((3, 7, 5))],
        [jax.ShapeDtypeStruct((n, 16), jnp.float32),
         jax.ShapeDtypeStruct((n, 3, 32), jnp.float32)],
        [_row_spec(16), _row3_spec(3, 32)])


def _stage_g(acc2p, d1, rdd2_root, rdd2_bias, n, acc_rows):
    def body(ap, d1r, root, biasr, out_o):
        acc = ap[0] + ap[1]
        out_o[...] = _rgcn_mix(acc, d1r[:, :7], root[...], biasr[...], 5)
    return _tc_call(
        body, n, [acc2p, d1, rdd2_root, rdd2_bias.reshape(1, -1)],
        [_part_spec(acc_rows, 32), _row_spec(16), _full_spec((7, 5)),
         _full_spec((1, 5))],
        jax.ShapeDtypeStruct((n, 5), jnp.float32), _row_spec(5))


def kernel(p_feat, d_feat, pp_edge_ind, pd_edge_ind, dd_edge_ind, dd_edge_type,
           W1, b1, W2, b2, rpd_basis, rpd_comp, rpd_root, rpd_bias,
           Wd, bd, rdd1_basis, rdd1_comp, rdd1_root, rdd1_bias,
           rdd2_basis, rdd2_comp, rdd2_root, rdd2_bias):
    n = p_feat.shape[0]
    acc_rows, _, _ = _acc_geometry(n)
    junk = n

    n_junk = acc_rows - n
    pp_src = _pad_idx(pp_edge_ind[0], 0, n)
    pp_dst = _pad_idx(pp_edge_ind[1], junk, n_junk)
    pd_src = _pad_idx(pd_edge_ind[0], 0, n)
    pd_dst = _pad_idx(pd_edge_ind[1], junk, n_junk)
    dd_g = _pad_idx(3 * dd_edge_ind[0] + dd_edge_type, 0, 3 * n)
    dd_dst = _pad_idx(dd_edge_ind[1], junk, n_junk)

    one_rows = jnp.concatenate(
        [jnp.ones((LANE, 1), jnp.float32),
         jnp.zeros((LANE, 15), jnp.float32)], axis=1)

    degp = _sc_count_pass(one_rows, pp_dst, n, 16, 8)
    xw1, d0 = _stage_a(p_feat, W1, d_feat, Wd, bd, n)

    y1 = _stage_b(degp, xw1, n, acc_rows)
    agg1p = _sc_gather_pass(y1, pp_src, pp_dst, n, 16, 8)

    y2, xw2 = _stage_c(degp, agg1p, xw1, W2, b1, n, acc_rows)
    agg2p = _sc_gather_pass(y2, pp_src, pp_dst, n, 16, 8)

    p2, y3 = _stage_d(degp, agg2p, xw2, b2, rpd_comp,
                      rpd_basis.reshape(5, 4), n, acc_rows)
    agg3p = _sc_gather_pass(y3, pd_src, pd_dst, n, 16, 8)

    x10, g1 = _stage_e(agg3p, p2, rpd_root, rpd_bias, d0,
                       rdd1_comp, rdd1_basis, n, acc_rows)
    acc1p = _sc_gather_pass(g1.reshape(3 * n, 32), dd_g, dd_dst, n, 32, 4)

    d1, g2 = _stage_f(acc1p, x10, rdd1_root, rdd1_bias,
                      rdd2_comp, rdd2_basis, n, acc_rows)
    acc2p = _sc_gather_pass(g2.reshape(3 * n, 32), dd_g, dd_dst, n, 32, 4)

    return _stage_g(acc2p, d1, rdd2_root, rdd2_bias, n, acc_rows)


# speedup vs baseline: 1.6232x; 1.0339x over previous
import functools

import jax
import jax.numpy as jnp
from jax import lax
from jax.experimental import pallas as pl
from jax.experimental.pallas import tpu as pltpu
from jax.experimental.pallas import tpu_sc as plsc

_SC_PARAMS = pltpu.CompilerParams(use_tc_tiling_on_sc=False,
                                  internal_scratch_in_bytes=256 * 1024)

NC = 2
NS = 16
NW = NC * NS
LANE = 128
IDX_BLK = 8
TILE = 2000


def _round_up(x, m):
    return (x + m - 1) // m * m


def _acc_geometry(n):
    slab = _round_up(-(-(n + 1) // NS), 64)
    zch = next(z for z in range(128, 7, -8) if slab % z == 0)
    return NS * slab, slab, zch


def _sc_gather_pass(g_tbl, idx_g, idx_s, n, width, ib):
    acc_rows, slab, zch = _acc_geometry(n)
    n_zch = slab // zch
    w_per = idx_g.shape[0] // NW
    n_blk = w_per // ib
    zeros = jnp.zeros((zch, width), jnp.float32)
    mesh = plsc.VectorSubcoreMesh(core_axis_name="c", subcore_axis_name="s")

    @functools.partial(
        pl.kernel, mesh=mesh, compiler_params=_SC_PARAMS,
        out_type=jax.ShapeDtypeStruct((NC, acc_rows, width), jnp.float32),
        scratch_types=[
            pltpu.VMEM_SHARED((acc_rows, width), jnp.float32),
            pltpu.VMEM((ib, LANE), jnp.int32),
            pltpu.VMEM((ib, LANE), jnp.int32),
            pltpu.VMEM((ib, LANE, width), jnp.float32),
            pltpu.VMEM((zch, width), jnp.float32),
            pltpu.SemaphoreType.DMA,
            pltpu.SemaphoreType.DMA,
        ])
    def run(g_hbm, ig_hbm, is_hbm, z_hbm, out_hbm,
            acc, igv, isv, rowsb, zbuf, semg, sems):
        cid = lax.axis_index("c")
        sid = lax.axis_index("s")
        wid = sid * NC + cid
        pltpu.sync_copy(z_hbm, zbuf)
        zd = [pltpu.async_copy(
            zbuf, acc.at[pl.ds((sid * n_zch + k) * zch, zch)], semg)
            for k in range(n_zch)]
        for d in zd:
            d.wait()
        plsc.subcore_barrier()
        wbase = wid * w_per

        @pl.loop(0, n_blk)
        def _(t):
            pltpu.sync_copy(ig_hbm.at[pl.ds(wbase + t * ib, ib)], igv)
            pltpu.sync_copy(is_hbm.at[pl.ds(wbase + t * ib, ib)], isv)
            gd = [pltpu.async_copy(g_hbm.at[igv.at[j]], rowsb.at[j], semg)
                  for j in range(ib)]
            for d in gd:
                d.wait()
            sd = [pltpu.async_copy(rowsb.at[j], acc.at[isv.at[j]], sems, add=True)
                  for j in range(ib)]
            for d in sd:
                d.wait()

        plsc.subcore_barrier()
        wd = [pltpu.async_copy(
            acc.at[pl.ds((sid * n_zch + k) * zch, zch)],
            out_hbm.at[cid, pl.ds((sid * n_zch + k) * zch, zch)], semg)
            for k in range(n_zch)]
        for d in wd:
            d.wait()

    return run(g_tbl, idx_g, idx_s, zeros)


def _sc_count_pass(const_rows, idx_s, n, width, ib):
    acc_rows, slab, zch = _acc_geometry(n)
    n_zch = slab // zch
    w_per = idx_s.shape[0] // NW
    n_blk = w_per // ib
    zeros = jnp.zeros((zch, width), jnp.float32)
    mesh = plsc.VectorSubcoreMesh(core_axis_name="c", subcore_axis_name="s")

    @functools.partial(
        pl.kernel, mesh=mesh, compiler_params=_SC_PARAMS,
        out_type=jax.ShapeDtypeStruct((NC, acc_rows, width), jnp.float32),
        scratch_types=[
            pltpu.VMEM_SHARED((acc_rows, width), jnp.float32),
            pltpu.VMEM((ib, LANE), jnp.int32),
            pltpu.VMEM((LANE, width), jnp.float32),
            pltpu.VMEM((zch, width), jnp.float32),
            pltpu.SemaphoreType.DMA,
        ])
    def run(c_hbm, is_hbm, z_hbm, out_hbm, acc, isv, rowsb, zbuf, sems):
        cid = lax.axis_index("c")
        sid = lax.axis_index("s")
        wid = sid * NC + cid
        pltpu.sync_copy(z_hbm, zbuf)
        pltpu.sync_copy(c_hbm, rowsb)
        zd = [pltpu.async_copy(
            zbuf, acc.at[pl.ds((sid * n_zch + k) * zch, zch)], sems)
            for k in range(n_zch)]
        for d in zd:
            d.wait()
        plsc.subcore_barrier()
        wbase = wid * w_per

        @pl.loop(0, n_blk)
        def _(t):
            pltpu.sync_copy(is_hbm.at[pl.ds(wbase + t * ib, ib)], isv)
            sd = [pltpu.async_copy(rowsb, acc.at[isv.at[j]], sems, add=True)
                  for j in range(ib)]
            for d in sd:
                d.wait()

        plsc.subcore_barrier()
        wd = [pltpu.async_copy(
            acc.at[pl.ds((sid * n_zch + k) * zch, zch)],
            out_hbm.at[cid, pl.ds((sid * n_zch + k) * zch, zch)], sems)
            for k in range(n_zch)]
        for d in wd:
            d.wait()

    return run(const_rows, idx_s, zeros)


def _pad_idx(idx, pad_base, pad_num):
    e = idx.shape[0]
    rows = _round_up(-(-e // LANE), NC * NS * IDX_BLK)
    pad = rows * LANE - e
    fill = pad_base + jnp.arange(pad, dtype=jnp.int32) % pad_num
    return jnp.concatenate([idx.astype(jnp.int32), fill]).reshape(rows, LANE)


def _row_spec(f):
    return pl.BlockSpec((TILE, f), lambda i: (i, 0))


def _row3_spec(a, b):
    return pl.BlockSpec((TILE, a, b), lambda i: (i, 0, 0))


def _part_spec(acc_rows, f):
    return pl.BlockSpec((NC, TILE, f), lambda i: (0, i, 0))


def _full_spec(shape):
    nd = len(shape)
    return pl.BlockSpec(shape, lambda i, _n=nd: (0,) * _n)


def _tc_call(body, n, in_arrays, in_specs, out_shapes, out_specs):
    grid = n // TILE
    return pl.pallas_call(
        body, grid=(grid,), in_specs=in_specs, out_specs=out_specs,
        out_shape=out_shapes)(*in_arrays)


def _pad16(x):
    return jnp.concatenate(
        [x, jnp.zeros((x.shape[0], 16 - x.shape[1]), x.dtype)], axis=1)


def _stage_a(p_feat, W1, d_feat, Wd, bd, n):
    def body(pf, w1, df, wd, bdr, xw1_o, d0_o):
        xw1_o[...] = _pad16(jnp.dot(pf[...], w1[...],
                                    preferred_element_type=jnp.float32))
        d0 = jnp.dot(df[...], wd[...], preferred_element_type=jnp.float32)
        d0_o[...] = _pad16(jnp.maximum(d0 + bdr[...], 0.0))
    return _tc_call(
        body, n,
        [p_feat, W1, d_feat, Wd, bd.reshape(1, -1)],
        [_row_spec(128), _full_spec(W1.shape), _row_spec(128),
         _full_spec(Wd.shape), _full_spec((1, bd.shape[0]))],
        [jax.ShapeDtypeStruct((n, 16), jnp.float32),
         jax.ShapeDtypeStruct((n, 16), jnp.float32)],
        [_row_spec(16), _row_spec(16)])


def _dinv_of(degp):
    deg = degp[0, :, 0:1] + degp[1, :, 0:1] + 1.0
    return lax.rsqrt(deg)


def _stage_b(degp, xw1, n, acc_rows):
    def body(dp, xw, y1_o):
        y1_o[...] = _dinv_of(dp[...]) * xw[...]
    return _tc_call(
        body, n, [degp, xw1], [_part_spec(acc_rows, 16), _row_spec(16)],
        jax.ShapeDtypeStruct((n, 16), jnp.float32), _row_spec(16))


def _stage_c(degp, agg1p, xw1, W2, b1, n, acc_rows):
    def body(dp, ap, xw, w2, b1r, y2_o, xw2_o):
        dinv = _dinv_of(dp[...])
        agg = ap[0] + ap[1]
        h1 = jnp.maximum(dinv * agg + dinv * dinv * xw[...] + b1r[...], 0.0)
        xw2 = _pad16(jnp.dot(h1[:, :10], w2[...],
                             preferred_element_type=jnp.float32))
        xw2_o[...] = xw2
        y2_o[...] = dinv * xw2
    b1p = _pad16(b1.reshape(1, -1))
    return _tc_call(
        body, n, [degp, agg1p, xw1, W2, b1p],
        [_part_spec(acc_rows, 16), _part_spec(acc_rows, 16), _row_spec(16),
         _full_spec(W2.shape), _full_spec((1, 16))],
        [jax.ShapeDtypeStruct((n, 16), jnp.float32),
         jax.ShapeDtypeStruct((n, 16), jnp.float32)],
        [_row_spec(16), _row_spec(16)])


def _stage_d(degp, agg2p, xw2, b2, rpd_comp, rpd_basis2d, n, acc_rows):
    def body(dp, ap, xw, b2r, comp, basis, p2_o, y3_o):
        dinv = _dinv_of(dp[...])
        agg = ap[0] + ap[1]
        p2 = jnp.maximum(dinv * agg + dinv * dinv * xw[...] + b2r[...], 0.0)
        p2_o[...] = p2
        wpd = comp[0, 0] * basis[...]
        zw = jnp.dot(p2[:, :5], wpd, preferred_element_type=jnp.float32)
        y3_o[...] = jnp.concatenate(
            [zw, jnp.ones((TILE, 1), jnp.float32),
             jnp.zeros((TILE, 11), jnp.float32)], axis=1)
    b2p = _pad16(b2.reshape(1, -1))
    return _tc_call(
        body, n, [degp, agg2p, xw2, b2p, rpd_comp, rpd_basis2d],
        [_part_spec(acc_rows, 16), _part_spec(acc_rows, 16), _row_spec(16),
         _full_spec((1, 16)), _full_spec((1, 1)), _full_spec((5, 4))],
        [jax.ShapeDtypeStruct((n, 16), jnp.float32),
         jax.ShapeDtypeStruct((n, 16), jnp.float32)],
        [_row_spec(16), _row_spec(16)])


def _rel_rows(a, r, count_col=7):
    f = a.shape[1]
    parts = [jnp.zeros((a.shape[0], 8 * r), jnp.float32), a,
             jnp.zeros((a.shape[0], count_col - f), jnp.float32),
             jnp.ones((a.shape[0], 1), jnp.float32),
             jnp.zeros((a.shape[0], 24 - 8 * r), jnp.float32)]
    return jnp.concatenate([p for p in parts if p.shape[1] > 0], axis=1)


def _stage_e(agg3p, p2, rpd_root, rpd_bias, d0, rdd1_comp, rdd1_basis,
             n, acc_rows):
    def body(ap, p2r, root, biasr, d0r, comp, basis, x10_o, g1_o):
        agg = ap[0] + ap[1]
        cnt = jnp.maximum(agg[:, 4:5], 1.0)
        p3 = jnp.maximum(
            jnp.dot(p2r[:, :5], root[...], preferred_element_type=jnp.float32)
            + biasr[...] + agg[:, :4] / cnt, 0.0)
        x10 = jnp.concatenate([d0r[:, :6], p3], axis=1)
        x10_o[...] = _pad16(x10)
        w = jnp.einsum('rb,bio->rio', comp[...], basis[...],
                       preferred_element_type=jnp.float32)
        rows = [_rel_rows(jnp.dot(x10, w[r],
                                  preferred_element_type=jnp.float32), r)
                for r in range(3)]
        g1_o[...] = jnp.stack(rows, axis=1)
    return _tc_call(
        body, n,
        [agg3p, p2, rpd_root, rpd_bias.reshape(1, -1), d0, rdd1_comp,
         rdd1_basis],
        [_part_spec(acc_rows, 16), _row_spec(16), _full_spec((5, 4)),
         _full_spec((1, 4)), _row_spec(16), _full_spec((3, 3)),
         _full_spec((3, 10, 7))],
        [jax.ShapeDtypeStruct((n, 16), jnp.float32),
         jax.ShapeDtypeStruct((n, 3, 32), jnp.float32)],
        [_row_spec(16), _row3_spec(3, 32)])


def _rgcn_mix(acc, x, root, bias, f_out):
    out = jnp.dot(x, root, preferred_element_type=jnp.float32) + bias
    for r in range(3):
        cnt = jnp.maximum(acc[:, 8 * r + 7:8 * r + 8], 1.0)
        out = out + acc[:, 8 * r:8 * r + f_out] / cnt
    return out


def _stage_f(acc1p, x10, rdd1_root, rdd1_bias, rdd2_comp, rdd2_basis,
             n, acc_rows):
    def body(ap, x10r, root, biasr, comp, basis, d1_o, g2_o):
        acc = ap[0] + ap[1]
        d1 = jnp.maximum(_rgcn_mix(acc, x10r[:, :10], root[...], biasr[...], 7),
                         0.0)
        d1_o[...] = _pad16(d1)
        w = jnp.einsum('rb,bio->rio', comp[...], basis[...],
                       preferred_element_type=jnp.float32)
        rows = [_rel_rows(jnp.dot(d1, w[r],
                                  preferred_element_type=jnp.float32), r)
                for r in range(3)]
        g2_o[...] = jnp.stack(rows, axis=1)
    return _tc_call(
        body, n,
        [acc1p, x10, rdd1_root, rdd1_bias.reshape(1, -1), rdd2_comp,
         rdd2_basis],
        [_part_spec(acc_rows, 32), _row_spec(16), _full_spec((10, 7)),
         _full_spec((1, 7)), _full_spec((3, 3)), _full_spec((3, 7, 5))],
        [jax.ShapeDtypeStruct((n, 16), jnp.float32),
         jax.ShapeDtypeStruct((n, 3, 32), jnp.float32)],
        [_row_spec(16), _row3_spec(3, 32)])


def _stage_g(acc2p, d1, rdd2_root, rdd2_bias, n, acc_rows):
    def body(ap, d1r, root, biasr, out_o):
        acc = ap[0] + ap[1]
        out_o[...] = _rgcn_mix(acc, d1r[:, :7], root[...], biasr[...], 5)
    return _tc_call(
        body, n, [acc2p, d1, rdd2_root, rdd2_bias.reshape(1, -1)],
        [_part_spec(acc_rows, 32), _row_spec(16), _full_spec((7, 5)),
         _full_spec((1, 5))],
        jax.ShapeDtypeStruct((n, 5), jnp.float32), _row_spec(5))


def kernel(p_feat, d_feat, pp_edge_ind, pd_edge_ind, dd_edge_ind, dd_edge_type,
           W1, b1, W2, b2, rpd_basis, rpd_comp, rpd_root, rpd_bias,
           Wd, bd, rdd1_basis, rdd1_comp, rdd1_root, rdd1_bias,
           rdd2_basis, rdd2_comp, rdd2_root, rdd2_bias):
    n = p_feat.shape[0]
    acc_rows, _, _ = _acc_geometry(n)
    junk = n

    n_junk = acc_rows - n
    pp_src = _pad_idx(pp_edge_ind[0], 0, n)
    pp_dst = _pad_idx(pp_edge_ind[1], junk, n_junk)
    pd_src = _pad_idx(pd_edge_ind[0], 0, n)
    pd_dst = _pad_idx(pd_edge_ind[1], junk, n_junk)
    dd_g = _pad_idx(3 * dd_edge_ind[0] + dd_edge_type, 0, 3 * n)
    dd_dst = _pad_idx(dd_edge_ind[1], junk, n_junk)

    one_rows = jnp.concatenate(
        [jnp.ones((LANE, 1), jnp.float32),
         jnp.zeros((LANE, 15), jnp.float32)], axis=1)

    degp = _sc_count_pass(one_rows, pp_dst, n, 16, 8)
    xw1, d0 = _stage_a(p_feat, W1, d_feat, Wd, bd, n)

    y1 = _stage_b(degp, xw1, n, acc_rows)
    agg1p = _sc_gather_pass(y1, pp_src, pp_dst, n, 16, 8)

    y2, xw2 = _stage_c(degp, agg1p, xw1, W2, b1, n, acc_rows)
    agg2p = _sc_gather_pass(y2, pp_src, pp_dst, n, 16, 8)

    p2, y3 = _stage_d(degp, agg2p, xw2, b2, rpd_comp,
                      rpd_basis.reshape(5, 4), n, acc_rows)
    agg3p = _sc_gather_pass(y3, pd_src, pd_dst, n, 16, 8)

    x10, g1 = _stage_e(agg3p, p2, rpd_root, rpd_bias, d0,
                       rdd1_comp, rdd1_basis, n, acc_rows)
    acc1p = _sc_gather_pass(g1.reshape(3 * n, 32), dd_g, dd_dst, n, 32, 4)

    d1, g2 = _stage_f(acc1p, x10, rdd1_root, rdd1_bias,
                      rdd2_comp, rdd2_basis, n, acc_rows)
    acc2p = _sc_gather_pass(g2.reshape(3 * n, 32), dd_g, dd_dst, n, 32, 4)

    return _stage_g(acc2p, d1, rdd2_root, rdd2_bias, n, acc_rows)
